# baseline (speedup 1.0000x reference)
import dataclasses
import functools

import jax
import jax.numpy as jnp
from jax import lax
from jax.experimental import pallas as pl
from jax.experimental.pallas import tpu as pltpu
from jax.experimental.pallas import tpu_sc as plsc

B = 16
IN_LEN = 50000
OUT_LEN = 5000
NNZ = 1600000
NCORES = 2
LANES = 16
HALF_LEN = 2560
OUT_PAD = 2 * HALF_LEN
CHUNK = 8192
STRIDE = CHUNK // LANES
UNROLL = 4


def _body(x_hbm, rows_hbm, cols_hbm, w_hbm, bias_hbm, off_hbm, out_hbm,
          xb, accv, rbuf0, cbuf0, wbuf0, rbuf1, cbuf1, wbuf1, offv, bbuf,
          sem_a, sem_b):
    c_idx = lax.axis_index("c")
    s_idx = lax.axis_index("s")
    wid = s_idx * NCORES + c_idx
    b = wid % B
    h = wid // B

    pltpu.sync_copy(x_hbm.at[b], xb)
    pltpu.sync_copy(off_hbm, offv)

    iot = lax.iota(jnp.int32, LANES)
    iot_s = iot * STRIDE
    ov = offv[...]
    n_lo = jnp.sum(jnp.where(iot == h, ov, 0))
    n_hi = jnp.sum(jnp.where(iot == h + 1, ov, 0))

    c0 = h * HALF_LEN
    pltpu.sync_copy(bias_hbm.at[pl.ds(c0, HALF_LEN)], bbuf)

    @pl.loop(0, HALF_LEN, step=LANES)
    def _init(j):
        accv[pl.ds(c0 + j, LANES)] = bbuf[pl.ds(j, LANES)]

    lo_al = n_lo & ~7
    nchunks = (n_hi - lo_al + CHUNK - 1) // CHUNK

    def chunk_base(k):
        start = lo_al + k * CHUNK
        return pl.multiple_of(jnp.minimum(start, NNZ - CHUNK), 8)

    def copies(k, rb, cb, wb, sem):
        base = chunk_base(k)
        return (
            pltpu.make_async_copy(rows_hbm.at[pl.ds(base, CHUNK)], rb, sem),
            pltpu.make_async_copy(cols_hbm.at[pl.ds(base, CHUNK)], cb, sem),
            pltpu.make_async_copy(w_hbm.at[pl.ds(base, CHUNK)], wb, sem),
        )

    def issue(k, rb, cb, wb, sem):
        for c in copies(k, rb, cb, wb, sem):
            c.start()

    def drain(k, rb, cb, wb, sem):
        for c in copies(k, rb, cb, wb, sem):
            c.wait()

    def compute(k, rb, cb, wb):
        start = lo_al + k * CHUNK
        base = chunk_base(k)
        interior = (start >= n_lo) & (start + CHUNK <= n_hi)

        @pl.when(interior)
        def _fast():
            @plsc.parallel_loop(0, STRIDE, unroll=UNROLL)
            def _grp(i):
                idx = iot_s + i
                r = plsc.load_gather(rb, [idx])
                cv = plsc.load_gather(cb, [idx])
                wv = plsc.load_gather(wb, [idx])
                xv = plsc.load_gather(xb, [r])
                plsc.addupdate_scatter(accv, [cv], xv * wv)

        @pl.when(jnp.logical_not(interior))
        def _masked():
            lb = jnp.maximum(n_lo, start)

            @plsc.parallel_loop(0, STRIDE, unroll=UNROLL)
            def _grp(i):
                idx = iot_s + i
                g = (base + i) + iot_s
                m = (g >= lb) & (g < n_hi)
                r = plsc.load_gather(rb, [idx])
                cv = plsc.load_gather(cb, [idx])
                wv = plsc.load_gather(wb, [idx])
                xv = plsc.load_gather(xb, [r])
                plsc.addupdate_scatter(accv, [cv], xv * wv, mask=m)

    issue(0, rbuf0, cbuf0, wbuf0, sem_a)
    npairs = (nchunks + 1) // 2

    def pair(p, carry):
        k0 = 2 * p
        drain(k0, rbuf0, cbuf0, wbuf0, sem_a)
        issue(k0 + 1, rbuf1, cbuf1, wbuf1, sem_b)
        compute(k0, rbuf0, cbuf0, wbuf0)
        drain(k0 + 1, rbuf1, cbuf1, wbuf1, sem_b)
        issue(k0 + 2, rbuf0, cbuf0, wbuf0, sem_a)
        compute(k0 + 1, rbuf1, cbuf1, wbuf1)
        return carry

    lax.fori_loop(0, npairs, pair, 0)
    drain(2 * npairs, rbuf0, cbuf0, wbuf0, sem_a)

    pltpu.sync_copy(accv.at[pl.ds(c0, HALF_LEN)],
                    out_hbm.at[b, pl.ds(c0, HALF_LEN)])


@jax.jit
def kernel(x, mask_rows, mask_cols, kernel, bias):
    x2 = x.reshape(B, IN_LEN)
    bias_pad = jnp.pad(bias[:, 0], (0, OUT_PAD - OUT_LEN))
    mid = jnp.searchsorted(mask_cols, HALF_LEN).astype(jnp.int32)
    off = jnp.zeros((LANES,), jnp.int32)
    off = off.at[1].set(mid)
    off = off.at[2:].set(NNZ)

    mesh = plsc.VectorSubcoreMesh(core_axis_name="c", subcore_axis_name="s")
    cp = pltpu.CompilerParams()
    if "needs_layout_passes" in pltpu.CompilerParams.__dataclass_fields__:
        cp = dataclasses.replace(cp, needs_layout_passes=False)
    run = functools.partial(
        pl.kernel,
        compiler_params=cp,
        out_type=jax.ShapeDtypeStruct((B, OUT_PAD), jnp.float32),
        mesh=mesh,
        scratch_types=[
            pltpu.VMEM((IN_LEN,), jnp.float32),
            pltpu.VMEM((OUT_PAD,), jnp.float32),
            pltpu.VMEM((CHUNK,), jnp.int32),
            pltpu.VMEM((CHUNK,), jnp.int32),
            pltpu.VMEM((CHUNK,), jnp.float32),
            pltpu.VMEM((CHUNK,), jnp.int32),
            pltpu.VMEM((CHUNK,), jnp.int32),
            pltpu.VMEM((CHUNK,), jnp.float32),
            pltpu.VMEM((LANES,), jnp.int32),
            pltpu.VMEM((HALF_LEN,), jnp.float32),
            pltpu.SemaphoreType.DMA,
            pltpu.SemaphoreType.DMA,
        ],
    )(_body)
    outp = run(x2, mask_rows, mask_cols, kernel, bias_pad, off)
    return outp[:, :OUT_LEN].reshape(B, OUT_LEN, 1)


# speedup vs baseline: 4.0707x; 4.0707x over previous
import dataclasses
import functools

import jax
import jax.numpy as jnp
from jax import lax
from jax.experimental import pallas as pl
from jax.experimental.pallas import tpu as pltpu
from jax.experimental.pallas import tpu_sc as plsc

B = 16
IN_LEN = 50000
OUT_LEN = 5000
NNZ = 1600000
NCORES = 2
LANES = 16
NQ = 4
QLEN = 1280
OUT_PAD = NQ * QLEN
SBLK = 8192
LSTRIDE = SBLK // LANES
NBLK = -(-NNZ // SBLK)
NNZ_PAD = NBLK * SBLK
CHUNK = 2048
UNROLL = 4


def _body(x_hbm, pk_hbm, w_hbm, bias_hbm, off_hbm, out_hbm,
          xb0, xb1, acc0, acc1, pbuf0, wbuf0, pbuf1, wbuf1, offv, bbuf,
          sem_a, sem_b):
    c_idx = lax.axis_index("c")
    s_idx = lax.axis_index("s")
    wid = s_idx * NCORES + c_idx
    bp = wid % (B // 2)
    q = wid // (B // 2)

    pltpu.sync_copy(x_hbm.at[2 * bp], xb0)
    pltpu.sync_copy(x_hbm.at[2 * bp + 1], xb1)
    pltpu.sync_copy(off_hbm, offv)

    iot = lax.iota(jnp.int32, LANES)
    iot_s = iot * LSTRIDE
    ov = offv[...]
    n_lo = jnp.sum(jnp.where(iot == q, ov, 0))
    n_hi = jnp.sum(jnp.where(iot == q + 1, ov, 0))

    c0 = q * QLEN
    pltpu.sync_copy(bias_hbm.at[pl.ds(c0, QLEN)], bbuf)

    @pl.loop(0, QLEN, step=LANES)
    def _init(j):
        bv = bbuf[pl.ds(j, LANES)]
        acc0[pl.ds(j, LANES)] = bv
        acc1[pl.ds(j, LANES)] = bv

    lo_al = n_lo & ~(CHUNK - 1)
    nchunks = (n_hi - lo_al + CHUNK - 1) // CHUNK

    def copies(k, pb, wb, sem):
        base = pl.multiple_of(
            jnp.minimum(lo_al + k * CHUNK, NNZ_PAD - CHUNK), CHUNK)
        return (
            pltpu.make_async_copy(pk_hbm.at[pl.ds(base, CHUNK)], pb, sem),
            pltpu.make_async_copy(w_hbm.at[pl.ds(base, CHUNK)], wb, sem),
        )

    def issue(k, pb, wb, sem):
        for c in copies(k, pb, wb, sem):
            c.start()

    def drain(k, pb, wb, sem):
        for c in copies(k, pb, wb, sem):
            c.wait()

    def compute(k, pb, wb):
        start = lo_al + k * CHUNK
        interior = (start >= n_lo) & (start + CHUNK <= n_hi)

        @pl.when(interior)
        def _fast():
            @plsc.parallel_loop(0, CHUNK, step=LANES, unroll=UNROLL)
            def _grp(j):
                sl = pl.ds(j, LANES)
                pk = pb[sl]
                r = pk & 0xFFFF
                cv = lax.shift_right_logical(pk, 16)
                wv = wb[sl]
                plsc.addupdate_scatter(
                    acc0, [cv], plsc.load_gather(xb0, [r]) * wv)
                plsc.addupdate_scatter(
                    acc1, [cv], plsc.load_gather(xb1, [r]) * wv)

        @pl.when(jnp.logical_not(interior))
        def _masked():
            lb = jnp.maximum(n_lo, start)
            gbase = (start & ~(SBLK - 1)) + ((start & (SBLK - 1)) >> 4)

            @plsc.parallel_loop(0, CHUNK, step=LANES, unroll=UNROLL)
            def _grp(j):
                g = (gbase + (j >> 4)) + iot_s
                m = (g >= lb) & (g < n_hi)
                sl = pl.ds(j, LANES)
                pk = pb[sl]
                r = pk & 0xFFFF
                cv = lax.shift_right_logical(pk, 16)
                wv = wb[sl]
                plsc.addupdate_scatter(
                    acc0, [cv], plsc.load_gather(xb0, [r]) * wv, mask=m)
                plsc.addupdate_scatter(
                    acc1, [cv], plsc.load_gather(xb1, [r]) * wv, mask=m)

    issue(0, pbuf0, wbuf0, sem_a)
    npairs = (nchunks + 1) // 2

    def pair(p, carry):
        k0 = 2 * p
        drain(k0, pbuf0, wbuf0, sem_a)
        issue(k0 + 1, pbuf1, wbuf1, sem_b)
        compute(k0, pbuf0, wbuf0)
        drain(k0 + 1, pbuf1, wbuf1, sem_b)
        issue(k0 + 2, pbuf0, wbuf0, sem_a)
        compute(k0 + 1, pbuf1, wbuf1)
        return carry

    lax.fori_loop(0, npairs, pair, 0)
    drain(2 * npairs, pbuf0, wbuf0, sem_a)

    pltpu.sync_copy(acc0, out_hbm.at[2 * bp, pl.ds(c0, QLEN)])
    pltpu.sync_copy(acc1, out_hbm.at[2 * bp + 1, pl.ds(c0, QLEN)])


def _stripe(a):
    a = jnp.pad(a, (0, NNZ_PAD - NNZ))
    return a.reshape(NBLK, LANES, LSTRIDE).transpose(0, 2, 1).reshape(-1)


@jax.jit
def kernel(x, mask_rows, mask_cols, kernel, bias):
    x2 = x.reshape(B, IN_LEN)
    bias_pad = jnp.pad(bias[:, 0], (0, OUT_PAD - OUT_LEN))
    qb = jnp.searchsorted(
        mask_cols, jnp.array([QLEN, 2 * QLEN, 3 * QLEN], jnp.int32)
    ).astype(jnp.int32)
    off = jnp.zeros((LANES,), jnp.int32)
    off = off.at[1:4].set(qb)
    off = off.at[4:].set(NNZ)

    local_col = mask_cols - (mask_cols // QLEN) * QLEN
    packed = jnp.bitwise_or(jnp.left_shift(local_col, 16), mask_rows)
    pk_p = _stripe(packed)
    w_p = _stripe(kernel)

    mesh = plsc.VectorSubcoreMesh(core_axis_name="c", subcore_axis_name="s")
    cp = pltpu.CompilerParams()
    if "needs_layout_passes" in pltpu.CompilerParams.__dataclass_fields__:
        cp = dataclasses.replace(cp, needs_layout_passes=False)
    run = functools.partial(
        pl.kernel,
        compiler_params=cp,
        out_type=jax.ShapeDtypeStruct((B, OUT_PAD), jnp.float32),
        mesh=mesh,
        scratch_types=[
            pltpu.VMEM((IN_LEN,), jnp.float32),
            pltpu.VMEM((IN_LEN,), jnp.float32),
            pltpu.VMEM((QLEN,), jnp.float32),
            pltpu.VMEM((QLEN,), jnp.float32),
            pltpu.VMEM((CHUNK,), jnp.int32),
            pltpu.VMEM((CHUNK,), jnp.float32),
            pltpu.VMEM((CHUNK,), jnp.int32),
            pltpu.VMEM((CHUNK,), jnp.float32),
            pltpu.VMEM((LANES,), jnp.int32),
            pltpu.VMEM((QLEN,), jnp.float32),
            pltpu.SemaphoreType.DMA,
            pltpu.SemaphoreType.DMA,
        ],
    )(_body)
    outp = run(x2, pk_p, w_p, bias_pad, off)
    return outp[:, :OUT_LEN].reshape(B, OUT_LEN, 1)
